# baseline (speedup 1.0000x reference)
import functools

import jax
import jax.numpy as jnp
from jax import lax
from jax.experimental import pallas as pl
from jax.experimental.pallas import tpu as pltpu
from jax.experimental.pallas import tpu_sc as plsc

_NC = 2
_NS = 16
_NW = _NC * _NS
_K = 16
_PACK = 4


def _fire(tabs, qvs, bigs, sems, chunk, buf):
    for t in range(2):
        pltpu.make_async_copy(
            tabs[t].at[qvs[t].at[chunk]], bigs[t].at[buf], sems[buf]).start()


def _drain(tabs, qvs, bigs, outs, sems, base, chunk, buf):
    for t in range(2):
        pltpu.make_async_copy(
            tabs[t].at[qvs[t].at[chunk]], bigs[t].at[buf], sems[buf]).wait()
    dst = pl.ds(base + chunk * _K, _K)
    for t in range(2):
        pltpu.sync_copy(bigs[t].at[buf], outs[t].at[dst])


def _gather_body(b_per_w,
                 uq_hbm, iq_hbm, utab_hbm, itab_hbm,
                 uout_hbm, iout_hbm,
                 uqv, iqv, ubig, ibig, sem0, sem1):
    wid = lax.axis_index("s") * _NC + lax.axis_index("c")
    base = wid * b_per_w
    pltpu.sync_copy(uq_hbm.at[wid], uqv)
    pltpu.sync_copy(iq_hbm.at[wid], iqv)
    n_chunks = b_per_w // _K
    tabs = (utab_hbm, itab_hbm)
    qvs = (uqv, iqv)
    bigs = (ubig, ibig)
    outs = (uout_hbm, iout_hbm)
    sems = (sem0, sem1)

    _fire(tabs, qvs, bigs, sems, 0, 0)
    for c in range(n_chunks // 2):
        _fire(tabs, qvs, bigs, sems, 2 * c + 1, 1)
        _drain(tabs, qvs, bigs, outs, sems, base, 2 * c, 0)
        if 2 * c + 2 < n_chunks:
            _fire(tabs, qvs, bigs, sems, 2 * c + 2, 0)
        _drain(tabs, qvs, bigs, outs, sems, base, 2 * c + 1, 1)


@functools.partial(jax.jit, static_argnums=(4,))
def _sc_gather(uq, iq, utab, itab, B):
    b_per_w = B // _NW
    n_chunks = b_per_w // _K
    mesh = plsc.VectorSubcoreMesh(core_axis_name="c", subcore_axis_name="s")
    body = functools.partial(_gather_body, b_per_w)
    kern = pl.kernel(
        body,
        out_type=[
            jax.ShapeDtypeStruct((B, 128), jnp.float32),
            jax.ShapeDtypeStruct((B, 128), jnp.float32),
        ],
        mesh=mesh,
        scratch_types=[
            pltpu.VMEM((n_chunks, _K), jnp.int32),
            pltpu.VMEM((n_chunks, _K), jnp.int32),
            pltpu.VMEM((2, _K, 128), jnp.float32),
            pltpu.VMEM((2, _K, 128), jnp.float32),
            pltpu.SemaphoreType.DMA,
            pltpu.SemaphoreType.DMA,
        ],
        compiler_params=pltpu.CompilerParams(needs_layout_passes=False),
    )
    shp = (_NW, n_chunks, _K)
    return kern(uq.reshape(shp), iq.reshape(shp), utab, itab)


def _select32(packed, off):
    r = packed[:, 96:128]
    r = jnp.where(off == 64, packed[:, 64:96], r)
    r = jnp.where(off == 32, packed[:, 32:64], r)
    r = jnp.where(off == 0, packed[:, 0:32], r)
    return r


def _mlp_body(u_ref, i_ref, uo_ref, io_ref, w1a_ref, w1b_ref, b1_ref,
              w2_ref, b2_ref, wp_ref, bp_ref, o_ref):
    u = _select32(u_ref[...], uo_ref[...])
    it = _select32(i_ref[...], io_ref[...])
    h1 = jnp.dot(u, w1a_ref[...], preferred_element_type=jnp.float32)
    h1 += jnp.dot(it, w1b_ref[...], preferred_element_type=jnp.float32)
    h1 = jnp.maximum(h1 + b1_ref[...], 0.0)
    h2 = jnp.dot(h1, w2_ref[...], preferred_element_type=jnp.float32)
    h2 = jnp.maximum(h2 + b2_ref[...], 0.0)
    p = jnp.dot(h2, wp_ref[...], preferred_element_type=jnp.float32)
    o_ref[...] = jax.nn.sigmoid(p + bp_ref[...])


def _tc_mlp(u, it, uo, io, W1, b1, W2, b2, Wp, bp, B, BK):
    w1a = W1[:, :32].T
    w1b = W1[:, 32:].T
    w2 = W2.T
    wp = Wp.T
    b1r = b1.reshape(1, -1)
    b2r = b2.reshape(1, -1)
    bpr = bp.reshape(1, -1)
    grid = B // BK

    def full(shape):
        return pl.BlockSpec(shape, lambda i: (0,) * len(shape))

    out = pl.pallas_call(
        _mlp_body,
        grid=(grid,),
        in_specs=[
            pl.BlockSpec((BK, 128), lambda i: (i, 0)),
            pl.BlockSpec((BK, 128), lambda i: (i, 0)),
            pl.BlockSpec((BK, 1), lambda i: (i, 0)),
            pl.BlockSpec((BK, 1), lambda i: (i, 0)),
            full(w1a.shape), full(w1b.shape), full(b1r.shape),
            full(w2.shape), full(b2r.shape),
            full(wp.shape), full(bpr.shape),
        ],
        out_specs=pl.BlockSpec((BK, 1), lambda i: (i, 0)),
        out_shape=jax.ShapeDtypeStruct((B, 1), jnp.float32),
    )(u, it, uo, io, w1a, w1b, b1r, w2, b2r, wp, bpr)
    return out


def kernel(user_indices, item_indices, user_emb, item_emb,
           W1, b1, W2, b2, Wp, bp):
    B = user_indices.shape[0]
    uidx = user_indices.astype(jnp.int32)
    iidx = item_indices.astype(jnp.int32)
    uq = uidx // _PACK
    iq = iidx // _PACK
    uo = ((uidx % _PACK) * 32).reshape(B, 1)
    io = ((iidx % _PACK) * 32).reshape(B, 1)
    utab = user_emb.reshape(user_emb.shape[0] // _PACK, 128)
    itab = item_emb.reshape(item_emb.shape[0] // _PACK, 128)
    u_rows, i_rows = _sc_gather(uq, iq, utab, itab, B)
    out = _tc_mlp(u_rows, i_rows, uo, io, W1, b1, W2, b2, Wp, bp, B, 2048)
    return jnp.squeeze(out, axis=-1)


# speedup vs baseline: 1.4973x; 1.4973x over previous
import functools

import jax
import jax.numpy as jnp
from jax import lax
from jax.experimental import pallas as pl
from jax.experimental.pallas import tpu as pltpu
from jax.experimental.pallas import tpu_sc as plsc

_NC = 2
_NS = 16
_NW = _NC * _NS
_K = 16


def _fire(tabs, qs, bigs, sems, chunk, buf):
    for t in range(2):
        idxs = qs[t][chunk]
        for k in range(_K):
            pltpu.make_async_copy(
                tabs[t].at[idxs[k]], bigs[t].at[buf].at[k],
                sems[buf]).start()


def _drain(tabs, qs, bigs, outs, sems, base, chunk, buf):
    for t in range(2):
        for k in range(_K):
            pltpu.make_async_copy(
                tabs[t].at[0], bigs[t].at[buf].at[k],
                sems[buf]).wait()
    dst = pl.ds(base + chunk * _K, _K)
    for t in range(2):
        pltpu.sync_copy(bigs[t].at[buf], outs[t].at[dst])


def _gather_body(b_per_w,
                 uq_hbm, iq_hbm, utab_hbm, itab_hbm,
                 uout_hbm, iout_hbm,
                 uqs, iqs, ubig, ibig, sem0, sem1):
    wid = lax.axis_index("s") * _NC + lax.axis_index("c")
    base = wid * b_per_w
    pltpu.sync_copy(uq_hbm.at[wid], uqs)
    pltpu.sync_copy(iq_hbm.at[wid], iqs)
    n_chunks = b_per_w // _K
    tabs = (utab_hbm, itab_hbm)
    qs = (uqs, iqs)
    bigs = (ubig, ibig)
    outs = (uout_hbm, iout_hbm)
    sems = (sem0, sem1)

    _fire(tabs, qs, bigs, sems, 0, 0)
    for c in range(n_chunks // 2):
        _fire(tabs, qs, bigs, sems, 2 * c + 1, 1)
        _drain(tabs, qs, bigs, outs, sems, base, 2 * c, 0)
        if 2 * c + 2 < n_chunks:
            _fire(tabs, qs, bigs, sems, 2 * c + 2, 0)
        _drain(tabs, qs, bigs, outs, sems, base, 2 * c + 1, 1)


@functools.partial(jax.jit, static_argnums=(4,))
def _sc_gather(uq, iq, utab, itab, B):
    b_per_w = B // _NW
    n_chunks = b_per_w // _K
    mesh = plsc.VectorSubcoreMesh(core_axis_name="c", subcore_axis_name="s")
    body = functools.partial(_gather_body, b_per_w)
    kern = pl.kernel(
        body,
        out_type=[
            jax.ShapeDtypeStruct((B, 32), jnp.float32),
            jax.ShapeDtypeStruct((B, 32), jnp.float32),
        ],
        mesh=mesh,
        scratch_types=[
            pltpu.VMEM((n_chunks, _K), jnp.int32),
            pltpu.VMEM((n_chunks, _K), jnp.int32),
            pltpu.VMEM((2, _K, 32), jnp.float32),
            pltpu.VMEM((2, _K, 32), jnp.float32),
            pltpu.SemaphoreType.DMA,
            pltpu.SemaphoreType.DMA,
        ],
        compiler_params=pltpu.CompilerParams(needs_layout_passes=False),
    )
    shp = (_NW, n_chunks, _K)
    return kern(uq.reshape(shp), iq.reshape(shp), utab, itab)


def _mlp_body(u_ref, i_ref, w1a_ref, w1b_ref, b1_ref,
              w2_ref, b2_ref, wp_ref, bp_ref, o_ref):
    h1 = jnp.dot(u_ref[...], w1a_ref[...], preferred_element_type=jnp.float32)
    h1 += jnp.dot(i_ref[...], w1b_ref[...], preferred_element_type=jnp.float32)
    h1 = jnp.maximum(h1 + b1_ref[...], 0.0)
    h2 = jnp.dot(h1, w2_ref[...], preferred_element_type=jnp.float32)
    h2 = jnp.maximum(h2 + b2_ref[...], 0.0)
    p = jnp.dot(h2, wp_ref[...], preferred_element_type=jnp.float32)
    o_ref[...] = jax.nn.sigmoid(p + bp_ref[...])


def _tc_mlp(u, it, W1, b1, W2, b2, Wp, bp, B, BK):
    w1a = W1[:, :32].T
    w1b = W1[:, 32:].T
    w2 = W2.T
    wp = Wp.T
    b1r = b1.reshape(1, -1)
    b2r = b2.reshape(1, -1)
    bpr = bp.reshape(1, -1)
    grid = B // BK

    def full(shape):
        return pl.BlockSpec(shape, lambda i: (0,) * len(shape))

    out = pl.pallas_call(
        _mlp_body,
        grid=(grid,),
        in_specs=[
            pl.BlockSpec((BK, 32), lambda i: (i, 0)),
            pl.BlockSpec((BK, 32), lambda i: (i, 0)),
            full(w1a.shape), full(w1b.shape), full(b1r.shape),
            full(w2.shape), full(b2r.shape),
            full(wp.shape), full(bpr.shape),
        ],
        out_specs=pl.BlockSpec((BK, 1), lambda i: (i, 0)),
        out_shape=jax.ShapeDtypeStruct((B, 1), jnp.float32),
    )(u, it, w1a, w1b, b1r, w2, b2r, wp, bpr)
    return out


def kernel(user_indices, item_indices, user_emb, item_emb,
           W1, b1, W2, b2, Wp, bp):
    B = user_indices.shape[0]
    uidx = user_indices.astype(jnp.int32)
    iidx = item_indices.astype(jnp.int32)
    u_rows, i_rows = _sc_gather(uidx, iidx, user_emb, item_emb, B)
    out = _tc_mlp(u_rows, i_rows, W1, b1, W2, b2, Wp, bp, B, 2048)
    return jnp.squeeze(out, axis=-1)


# speedup vs baseline: 1.5144x; 1.0114x over previous
import functools

import jax
import jax.numpy as jnp
from jax import lax
from jax.experimental import pallas as pl
from jax.experimental.pallas import tpu as pltpu
from jax.experimental.pallas import tpu_sc as plsc

_NC = 2
_NS = 16
_NW = _NC * _NS
_K = 32


def _fire(tabs, qs, bigs, sems, chunk, buf):
    for t in range(2):
        for h in range(_K // 16):
            idxs = qs[t][chunk, h * 16:(h + 1) * 16]
            for k in range(16):
                pltpu.make_async_copy(
                    tabs[t].at[idxs[k]], bigs[t].at[buf].at[h * 16 + k],
                    sems[buf]).start()


def _drain(tabs, qs, bigs, outs, sems, base, chunk, buf):
    for t in range(2):
        for k in range(_K):
            pltpu.make_async_copy(
                tabs[t].at[0], bigs[t].at[buf].at[k],
                sems[buf]).wait()
    dst = pl.ds(base + chunk * _K, _K)
    for t in range(2):
        pltpu.sync_copy(bigs[t].at[buf], outs[t].at[dst])


def _gather_body(b_per_w,
                 uq_hbm, iq_hbm, utab_hbm, itab_hbm,
                 uout_hbm, iout_hbm,
                 uqs, iqs, ubig, ibig, sem0, sem1):
    wid = lax.axis_index("s") * _NC + lax.axis_index("c")
    base = wid * b_per_w
    pltpu.sync_copy(uq_hbm.at[wid], uqs)
    pltpu.sync_copy(iq_hbm.at[wid], iqs)
    n_chunks = b_per_w // _K
    tabs = (utab_hbm, itab_hbm)
    qs = (uqs, iqs)
    bigs = (ubig, ibig)
    outs = (uout_hbm, iout_hbm)
    sems = (sem0, sem1)

    _fire(tabs, qs, bigs, sems, 0, 0)
    for c in range(n_chunks // 2):
        _fire(tabs, qs, bigs, sems, 2 * c + 1, 1)
        _drain(tabs, qs, bigs, outs, sems, base, 2 * c, 0)
        if 2 * c + 2 < n_chunks:
            _fire(tabs, qs, bigs, sems, 2 * c + 2, 0)
        _drain(tabs, qs, bigs, outs, sems, base, 2 * c + 1, 1)


@functools.partial(jax.jit, static_argnums=(4,))
def _sc_gather(uq, iq, utab, itab, B):
    b_per_w = B // _NW
    n_chunks = b_per_w // _K
    mesh = plsc.VectorSubcoreMesh(core_axis_name="c", subcore_axis_name="s")
    body = functools.partial(_gather_body, b_per_w)
    kern = pl.kernel(
        body,
        out_type=[
            jax.ShapeDtypeStruct((B, 32), jnp.float32),
            jax.ShapeDtypeStruct((B, 32), jnp.float32),
        ],
        mesh=mesh,
        scratch_types=[
            pltpu.VMEM((n_chunks, _K), jnp.int32),
            pltpu.VMEM((n_chunks, _K), jnp.int32),
            pltpu.VMEM((2, _K, 32), jnp.float32),
            pltpu.VMEM((2, _K, 32), jnp.float32),
            pltpu.SemaphoreType.DMA,
            pltpu.SemaphoreType.DMA,
        ],
        compiler_params=pltpu.CompilerParams(needs_layout_passes=False),
    )
    shp = (_NW, n_chunks, _K)
    return kern(uq.reshape(shp), iq.reshape(shp), utab, itab)


def _mlp_body(u_ref, i_ref, w1a_ref, w1b_ref, b1_ref,
              w2_ref, b2_ref, wp_ref, bp_ref, o_ref):
    h1 = jnp.dot(u_ref[...], w1a_ref[...], preferred_element_type=jnp.float32)
    h1 += jnp.dot(i_ref[...], w1b_ref[...], preferred_element_type=jnp.float32)
    h1 = jnp.maximum(h1 + b1_ref[...], 0.0)
    h2 = jnp.dot(h1, w2_ref[...], preferred_element_type=jnp.float32)
    h2 = jnp.maximum(h2 + b2_ref[...], 0.0)
    p = jnp.dot(h2, wp_ref[...], preferred_element_type=jnp.float32)
    o_ref[...] = jax.nn.sigmoid(p + bp_ref[...])


def _tc_mlp(u, it, W1, b1, W2, b2, Wp, bp, B, BK):
    w1a = W1[:, :32].T
    w1b = W1[:, 32:].T
    w2 = W2.T
    wp = Wp.T
    b1r = b1.reshape(1, -1)
    b2r = b2.reshape(1, -1)
    bpr = bp.reshape(1, -1)
    grid = B // BK

    def full(shape):
        return pl.BlockSpec(shape, lambda i: (0,) * len(shape))

    out = pl.pallas_call(
        _mlp_body,
        grid=(grid,),
        in_specs=[
            pl.BlockSpec((BK, 32), lambda i: (i, 0)),
            pl.BlockSpec((BK, 32), lambda i: (i, 0)),
            full(w1a.shape), full(w1b.shape), full(b1r.shape),
            full(w2.shape), full(b2r.shape),
            full(wp.shape), full(bpr.shape),
        ],
        out_specs=pl.BlockSpec((BK, 1), lambda i: (i, 0)),
        out_shape=jax.ShapeDtypeStruct((B, 1), jnp.float32),
    )(u, it, w1a, w1b, b1r, w2, b2r, wp, bpr)
    return out


def kernel(user_indices, item_indices, user_emb, item_emb,
           W1, b1, W2, b2, Wp, bp):
    B = user_indices.shape[0]
    uidx = user_indices.astype(jnp.int32)
    iidx = item_indices.astype(jnp.int32)
    u_rows, i_rows = _sc_gather(uidx, iidx, user_emb, item_emb, B)
    out = _tc_mlp(u_rows, i_rows, W1, b1, W2, b2, Wp, bp, B, 2048)
    return jnp.squeeze(out, axis=-1)


# speedup vs baseline: 1.5158x; 1.0009x over previous
import functools

import jax
import jax.numpy as jnp
from jax import lax
from jax.experimental import pallas as pl
from jax.experimental.pallas import tpu as pltpu
from jax.experimental.pallas import tpu_sc as plsc

_NC = 2
_NS = 16
_NW = _NC * _NS
_K = 64


def _fire(tabs, qs, bigs, sems, chunk, buf):
    for t in range(2):
        for h in range(_K // 16):
            idxs = qs[t][chunk, h * 16:(h + 1) * 16]
            for k in range(16):
                pltpu.make_async_copy(
                    tabs[t].at[idxs[k]], bigs[t].at[buf].at[h * 16 + k],
                    sems[buf]).start()


def _drain(tabs, qs, bigs, outs, sems, base, chunk, buf):
    for t in range(2):
        for k in range(_K):
            pltpu.make_async_copy(
                tabs[t].at[0], bigs[t].at[buf].at[k],
                sems[buf]).wait()
    dst = pl.ds(base + chunk * _K, _K)
    for t in range(2):
        pltpu.sync_copy(bigs[t].at[buf], outs[t].at[dst])


def _gather_body(b_per_w,
                 uq_hbm, iq_hbm, utab_hbm, itab_hbm,
                 uout_hbm, iout_hbm,
                 uqs, iqs, ubig, ibig, sem0, sem1):
    wid = lax.axis_index("s") * _NC + lax.axis_index("c")
    base = wid * b_per_w
    pltpu.sync_copy(uq_hbm.at[wid], uqs)
    pltpu.sync_copy(iq_hbm.at[wid], iqs)
    n_chunks = b_per_w // _K
    tabs = (utab_hbm, itab_hbm)
    qs = (uqs, iqs)
    bigs = (ubig, ibig)
    outs = (uout_hbm, iout_hbm)
    sems = (sem0, sem1)

    _fire(tabs, qs, bigs, sems, 0, 0)
    for c in range(n_chunks // 2):
        _fire(tabs, qs, bigs, sems, 2 * c + 1, 1)
        _drain(tabs, qs, bigs, outs, sems, base, 2 * c, 0)
        if 2 * c + 2 < n_chunks:
            _fire(tabs, qs, bigs, sems, 2 * c + 2, 0)
        _drain(tabs, qs, bigs, outs, sems, base, 2 * c + 1, 1)


@functools.partial(jax.jit, static_argnums=(4,))
def _sc_gather(uq, iq, utab, itab, B):
    b_per_w = B // _NW
    n_chunks = b_per_w // _K
    mesh = plsc.VectorSubcoreMesh(core_axis_name="c", subcore_axis_name="s")
    body = functools.partial(_gather_body, b_per_w)
    kern = pl.kernel(
        body,
        out_type=[
            jax.ShapeDtypeStruct((B, 32), jnp.float32),
            jax.ShapeDtypeStruct((B, 32), jnp.float32),
        ],
        mesh=mesh,
        scratch_types=[
            pltpu.VMEM((n_chunks, _K), jnp.int32),
            pltpu.VMEM((n_chunks, _K), jnp.int32),
            pltpu.VMEM((2, _K, 32), jnp.float32),
            pltpu.VMEM((2, _K, 32), jnp.float32),
            pltpu.SemaphoreType.DMA,
            pltpu.SemaphoreType.DMA,
        ],
        compiler_params=pltpu.CompilerParams(needs_layout_passes=False),
    )
    shp = (_NW, n_chunks, _K)
    return kern(uq.reshape(shp), iq.reshape(shp), utab, itab)


def _mlp_body(u_ref, i_ref, w1a_ref, w1b_ref, b1_ref,
              w2_ref, b2_ref, wp_ref, bp_ref, o_ref):
    h1 = jnp.dot(u_ref[...], w1a_ref[...], preferred_element_type=jnp.float32)
    h1 += jnp.dot(i_ref[...], w1b_ref[...], preferred_element_type=jnp.float32)
    h1 = jnp.maximum(h1 + b1_ref[...], 0.0)
    h2 = jnp.dot(h1, w2_ref[...], preferred_element_type=jnp.float32)
    h2 = jnp.maximum(h2 + b2_ref[...], 0.0)
    p = jnp.dot(h2, wp_ref[...], preferred_element_type=jnp.float32)
    o_ref[...] = jax.nn.sigmoid(p + bp_ref[...])


def _tc_mlp(u, it, W1, b1, W2, b2, Wp, bp, B, BK):
    w1a = W1[:, :32].T
    w1b = W1[:, 32:].T
    w2 = W2.T
    wp = Wp.T
    b1r = b1.reshape(1, -1)
    b2r = b2.reshape(1, -1)
    bpr = bp.reshape(1, -1)
    grid = B // BK

    def full(shape):
        return pl.BlockSpec(shape, lambda i: (0,) * len(shape))

    out = pl.pallas_call(
        _mlp_body,
        grid=(grid,),
        in_specs=[
            pl.BlockSpec((BK, 32), lambda i: (i, 0)),
            pl.BlockSpec((BK, 32), lambda i: (i, 0)),
            full(w1a.shape), full(w1b.shape), full(b1r.shape),
            full(w2.shape), full(b2r.shape),
            full(wp.shape), full(bpr.shape),
        ],
        out_specs=pl.BlockSpec((BK, 1), lambda i: (i, 0)),
        out_shape=jax.ShapeDtypeStruct((B, 1), jnp.float32),
    )(u, it, w1a, w1b, b1r, w2, b2r, wp, bpr)
    return out


def kernel(user_indices, item_indices, user_emb, item_emb,
           W1, b1, W2, b2, Wp, bp):
    B = user_indices.shape[0]
    uidx = user_indices.astype(jnp.int32)
    iidx = item_indices.astype(jnp.int32)
    u_rows, i_rows = _sc_gather(uidx, iidx, user_emb, item_emb, B)
    out = _tc_mlp(u_rows, i_rows, W1, b1, W2, b2, Wp, bp, B, 2048)
    return jnp.squeeze(out, axis=-1)
